# baseline (speedup 1.0000x reference)
import functools

import jax
import jax.numpy as jnp
from jax import lax
from jax.experimental import pallas as pl
from jax.experimental.pallas import tpu as pltpu
from jax.experimental.pallas import tpu_sc as plsc

D = 64
NC = 2
NS = 16
NW = NC * NS
CHUNK = 128
GRP = 16
NQ = D // GRP
UNROLL = 4


def _rsqrt_nr(s):
    i = lax.bitcast_convert_type(s, jnp.int32)
    i = jnp.int32(0x5F3759DF) - lax.shift_right_logical(i, 1)
    y = lax.bitcast_convert_type(i, jnp.float32)
    half_s = jnp.float32(0.5) * s
    for _ in range(3):
        y = y * (jnp.float32(1.5) - half_s * y * y)
    return y


@functools.partial(jax.jit, static_argnums=(2,))
def _gather_normalize(table, idx3d, b_each):
    n_chunks = idx3d.shape[1]
    w_per_arr = NW // 4
    mesh = plsc.VectorSubcoreMesh(core_axis_name="c", subcore_axis_name="s")
    out_sds = jax.ShapeDtypeStruct((b_each, D), jnp.float32)

    @functools.partial(
        pl.kernel,
        mesh=mesh,
        out_type=(out_sds, out_sds, out_sds, out_sds),
        compiler_params=pltpu.CompilerParams(use_tc_tiling_on_sc=False),
        scratch_types=[
            pltpu.VMEM((n_chunks, CHUNK), jnp.int32),
            pltpu.VMEM((CHUNK, D), jnp.float32),
            pltpu.VMEM((UNROLL * 2 * GRP,), jnp.float32),
            pltpu.SemaphoreType.DMA,
        ],
    )
    def k(table_hbm, idx_hbm, o0, o1, o2, o3, idx_v, buf, scr, sem):
        wid = lax.axis_index("s") * NC + lax.axis_index("c")
        aid = wid // w_per_arr
        arow = (wid % w_per_arr) * (n_chunks * CHUNK)
        pltpu.sync_copy(idx_hbm.at[wid], idx_v)

        def body(c, carry):
            pltpu.async_copy(table_hbm.at[idx_v.at[c]], buf, sem).wait()

            def rows_body(i, carry2):
                for u in range(UNROLL):
                    r = i * UNROLL + u
                    base = u * 2 * GRP
                    vs = [buf[r, pl.ds(q * GRP, GRP)] for q in range(NQ)]
                    s = jnp.zeros((GRP,), jnp.float32)
                    for v in vs:
                        s = s + v * v
                    scr[pl.ds(base, GRP)] = s
                    cur = s
                    for off in (8, 4, 2, 1):
                        cur = cur + scr[pl.ds(base + off, GRP)]
                        scr[pl.ds(base, GRP)] = cur
                    t = jnp.maximum(cur[0], jnp.float32(1e-24))
                    rs = jnp.full((GRP,), _rsqrt_nr(t), jnp.float32)
                    for q, v in enumerate(vs):
                        buf[r, pl.ds(q * GRP, GRP)] = v * rs
                return carry2

            lax.fori_loop(0, CHUNK // UNROLL, rows_body, 0)
            dst_row = arow + c * CHUNK
            for a, out in enumerate((o0, o1, o2, o3)):
                @pl.when(aid == a)
                def _():
                    pltpu.sync_copy(buf, out.at[pl.ds(dst_row, CHUNK)])
            return carry

        lax.fori_loop(0, n_chunks, body, 0)

    return k(table, idx3d)


def kernel(x1, adj, pos_src, pos_dst, neg_src, neg_dst):
    del adj
    b_each = pos_src.shape[0]
    idx = jnp.concatenate([
        pos_src.astype(jnp.int32), pos_dst.astype(jnp.int32),
        neg_src.astype(jnp.int32), neg_dst.astype(jnp.int32),
    ])
    per_w = (4 * b_each) // NW
    idx3d = idx.reshape(NW, per_w // CHUNK, CHUNK)
    return _gather_normalize(x1, idx3d, b_each)


# speedup vs baseline: 1.1384x; 1.1384x over previous
import functools

import jax
import jax.numpy as jnp
from jax import lax
from jax.experimental import pallas as pl
from jax.experimental.pallas import tpu as pltpu
from jax.experimental.pallas import tpu_sc as plsc

D = 64
NC = 2
NS = 16
NW = NC * NS
CHUNK = 128
GRP = 16
NQ = D // GRP
UNROLL = 4


def _rsqrt_nr(s):
    i = lax.bitcast_convert_type(s, jnp.int32)
    i = jnp.int32(0x5F3759DF) - lax.shift_right_logical(i, 1)
    y = lax.bitcast_convert_type(i, jnp.float32)
    half_s = jnp.float32(0.5) * s
    for _ in range(3):
        y = y * (jnp.float32(1.5) - half_s * y * y)
    return y


@functools.partial(jax.jit, static_argnums=(2,))
def _gather_normalize(table, idx3d, b_each):
    n_chunks = idx3d.shape[1]
    n_pairs = n_chunks // 2
    w_per_arr = NW // 4
    mesh = plsc.VectorSubcoreMesh(core_axis_name="c", subcore_axis_name="s")
    out_sds = jax.ShapeDtypeStruct((b_each, D), jnp.float32)

    @functools.partial(
        pl.kernel,
        mesh=mesh,
        out_type=(out_sds, out_sds, out_sds, out_sds),
        compiler_params=pltpu.CompilerParams(use_tc_tiling_on_sc=False),
        scratch_types=[
            pltpu.VMEM((n_chunks, CHUNK), jnp.int32),
            pltpu.VMEM((CHUNK, D), jnp.float32),
            pltpu.VMEM((CHUNK, D), jnp.float32),
            pltpu.SemaphoreType.DMA,
            pltpu.SemaphoreType.DMA,
        ],
    )
    def k(table_hbm, idx_hbm, o0, o1, o2, o3, idx_v, b0, b1, g0, g1):
        wid = lax.axis_index("s") * NC + lax.axis_index("c")
        aid = wid // w_per_arr
        arow = (wid % w_per_arr) * (n_chunks * CHUNK)
        pltpu.sync_copy(idx_hbm.at[wid], idx_v)

        lane = lax.iota(jnp.int32, GRP)
        perms = [lax.bitwise_xor(lane, jnp.int32(off)) for off in (8, 4, 2, 1)]

        def normalize(buf):
            def rows_body(i, carry):
                for u in range(UNROLL):
                    r = i * UNROLL + u
                    vs = [buf[r, pl.ds(q * GRP, GRP)] for q in range(NQ)]
                    s = jnp.zeros((GRP,), jnp.float32)
                    for v in vs:
                        s = s + v * v
                    for p in perms:
                        s = s + jnp.take_along_axis(
                            s, p, axis=0, mode="promise_in_bounds"
                        )
                    rs = _rsqrt_nr(jnp.maximum(s, jnp.float32(1e-24)))
                    for q, v in enumerate(vs):
                        buf[r, pl.ds(q * GRP, GRP)] = v * rs
                return carry

            lax.fori_loop(0, CHUNK // UNROLL, rows_body, 0)

        def write_out(buf, c):
            dst_row = arow + c * CHUNK
            for a, out in enumerate((o0, o1, o2, o3)):
                @pl.when(aid == a)
                def _():
                    pltpu.sync_copy(buf, out.at[pl.ds(dst_row, CHUNK)])

        pltpu.async_copy(table_hbm.at[idx_v.at[0]], b0, g0)

        def body(i, carry):
            c0 = 2 * i
            c1 = 2 * i + 1
            pltpu.make_async_copy(table_hbm.at[idx_v.at[c0]], b0, g0).wait()
            pltpu.async_copy(table_hbm.at[idx_v.at[c1]], b1, g1)
            normalize(b0)
            write_out(b0, c0)
            pltpu.make_async_copy(table_hbm.at[idx_v.at[c1]], b1, g1).wait()

            @pl.when(i + 1 < n_pairs)
            def _():
                pltpu.async_copy(table_hbm.at[idx_v.at[c0 + 2]], b0, g0)

            normalize(b1)
            write_out(b1, c1)
            return carry

        lax.fori_loop(0, n_pairs, body, 0)

    return k(table, idx3d)


def kernel(x1, adj, pos_src, pos_dst, neg_src, neg_dst):
    del adj
    b_each = pos_src.shape[0]
    idx = jnp.concatenate([
        pos_src.astype(jnp.int32), pos_dst.astype(jnp.int32),
        neg_src.astype(jnp.int32), neg_dst.astype(jnp.int32),
    ])
    per_w = (4 * b_each) // NW
    idx3d = idx.reshape(NW, per_w // CHUNK, CHUNK)
    return _gather_normalize(x1, idx3d, b_each)
